# baseline (speedup 1.0000x reference)
import functools

import jax
import jax.numpy as jnp
from jax import lax
from jax.experimental import pallas as pl
from jax.experimental.pallas import tpu as pltpu
from jax.experimental.pallas import tpu_sc as plsc

EMBED_DIM = 32
HALF_DIM = 16
NUM_WORKERS = 32
CHUNK_B = 8
LANES = 16
STAGE_HALVES = 2


def _sc_embedding_gather(x, table2, S):
    B, Sp = x.shape
    S2 = 2 * S
    b_per_w = B // NUM_WORKERS
    n_chunks = b_per_w // CHUNK_B
    half = b_per_w // STAGE_HALVES

    mesh = plsc.VectorSubcoreMesh(core_axis_name="c", subcore_axis_name="s")

    @functools.partial(
        pl.kernel,
        out_type=jax.ShapeDtypeStruct((B, S2, HALF_DIM), jnp.float32),
        mesh=mesh,
        scratch_types=[
            pltpu.VMEM((half, Sp), jnp.int32),
            pltpu.VMEM((b_per_w, S2), jnp.int32),
            pltpu.VMEM((2, CHUNK_B, S2, HALF_DIM), jnp.float32),
            pltpu.SemaphoreType.DMA,
            pltpu.SemaphoreType.DMA,
        ],
        compiler_params=pltpu.CompilerParams(
            use_tc_tiling_on_sc=False, needs_layout_passes=False
        ),
    )
    def body(x_hbm, tab_hbm, out_hbm, raw_v, idx_v, rows_v, gsem, ssem):
        wid = lax.axis_index("s") * 2 + lax.axis_index("c")
        base = wid * b_per_w

        lane = lax.iota(jnp.int32, LANES)
        for h in range(STAGE_HALVES):
            pltpu.sync_copy(x_hbm.at[pl.ds(base + h * half, half)], raw_v)

            def compact_body(r, carry, h=h):
                row = jnp.full((LANES,), h * half + r, jnp.int32)
                for k in range((S + LANES - 1) // LANES):
                    col = k * LANES + lane
                    v = raw_v[r, pl.ds(k * LANES, LANES)]
                    ok = col < S
                    plsc.store_scatter(idx_v, [row, 2 * col], 2 * v, mask=ok)
                    plsc.store_scatter(idx_v, [row, 2 * col + 1], 2 * v + 1, mask=ok)
                return carry

            lax.fori_loop(0, half, compact_body, 0)

        def fire_gather(c, slot):
            for r in range(CHUNK_B):
                pltpu.make_async_copy(
                    tab_hbm.at[idx_v.at[c * CHUNK_B + r]],
                    rows_v.at[slot, r],
                    gsem,
                ).start()

        def wait_gather(c, slot):
            for r in range(CHUNK_B):
                pltpu.make_async_copy(
                    tab_hbm.at[idx_v.at[c * CHUNK_B + r]],
                    rows_v.at[slot, r],
                    gsem,
                ).wait()

        def store_desc(c, slot):
            return pltpu.make_async_copy(
                rows_v.at[slot],
                out_hbm.at[pl.ds(base + c * CHUNK_B, CHUNK_B)],
                ssem,
            )

        fire_gather(0, 0)

        def chunk_body(c, carry):
            slot = lax.rem(c, 2)
            wait_gather(c, slot)

            @pl.when(c >= 1)
            def _():
                store_desc(c - 1, 1 - slot).wait()

            @pl.when(c + 1 < n_chunks)
            def _():
                fire_gather(c + 1, 1 - slot)

            store_desc(c, slot).start()
            return carry

        lax.fori_loop(0, n_chunks, chunk_body, 0)
        store_desc(n_chunks - 1, lax.rem(n_chunks - 1, 2)).wait()

    return body(x, table2)


def kernel(x, raw_weight):
    B, S = x.shape
    xp = jnp.pad(x.astype(jnp.int32), ((0, 0), (0, 128 - S)))
    table2 = jnp.reshape(raw_weight, (-1, HALF_DIM))
    out = _sc_embedding_gather(xp, table2, S)
    return jnp.reshape(out, (B, S, EMBED_DIM))


# speedup vs baseline: 1.4422x; 1.4422x over previous
import functools

import jax
import jax.numpy as jnp
from jax import lax
from jax.experimental import pallas as pl
from jax.experimental.pallas import tpu as pltpu
from jax.experimental.pallas import tpu_sc as plsc

EMBED_DIM = 32
NUM_WORKERS = 32
CHUNK_B = 8
LANES = 16
SP = 128
STAGE_HALVES = 2


def _sc_embedding_gather(x_flat, table, S, B):
    b_per_w = B // NUM_WORKERS
    n_chunks = b_per_w // CHUNK_B
    half = b_per_w // STAGE_HALVES

    mesh = plsc.VectorSubcoreMesh(core_axis_name="c", subcore_axis_name="s")

    @functools.partial(
        pl.kernel,
        out_type=jax.ShapeDtypeStruct((B, S, EMBED_DIM), jnp.float32),
        mesh=mesh,
        scratch_types=[
            pltpu.VMEM((half * SP,), jnp.int32),
            pltpu.VMEM((b_per_w, S), jnp.int32),
            pltpu.VMEM((2, CHUNK_B, S, EMBED_DIM), jnp.float32),
            pltpu.SemaphoreType.DMA,
            pltpu.SemaphoreType.DMA,
        ],
        compiler_params=pltpu.CompilerParams(
            use_tc_tiling_on_sc=False, needs_layout_passes=False
        ),
    )
    def body(x_hbm, tab_hbm, out_hbm, raw_v, idx_v, rows_v, gsem, ssem):
        wid = lax.axis_index("s") * 2 + lax.axis_index("c")
        base = wid * b_per_w

        lane = lax.iota(jnp.int32, LANES)
        for h in range(STAGE_HALVES):
            pltpu.sync_copy(
                x_hbm.at[pl.ds((base + h * half) * SP, half * SP)], raw_v
            )

            def compact_body(r, carry, h=h):
                row = jnp.full((LANES,), h * half + r, jnp.int32)
                for k in range((S + LANES - 1) // LANES):
                    col = k * LANES + lane
                    v = raw_v[pl.ds(r * SP + k * LANES, LANES)]
                    plsc.store_scatter(idx_v, [row, col], v, mask=col < S)
                return carry

            lax.fori_loop(0, half, compact_body, 0)

        def fire_gather(c, slot):
            for r in range(CHUNK_B):
                pltpu.make_async_copy(
                    tab_hbm.at[idx_v.at[c * CHUNK_B + r]],
                    rows_v.at[slot, r],
                    gsem,
                ).start()

        def wait_gather(c, slot):
            for r in range(CHUNK_B):
                pltpu.make_async_copy(
                    tab_hbm.at[idx_v.at[c * CHUNK_B + r]],
                    rows_v.at[slot, r],
                    gsem,
                ).wait()

        def store_desc(c, slot):
            return pltpu.make_async_copy(
                rows_v.at[slot],
                out_hbm.at[pl.ds(base + c * CHUNK_B, CHUNK_B)],
                ssem,
            )

        fire_gather(0, 0)

        def chunk_body(c, carry):
            slot = lax.rem(c, 2)
            wait_gather(c, slot)

            @pl.when(c >= 1)
            def _():
                store_desc(c - 1, 1 - slot).wait()

            @pl.when(c + 1 < n_chunks)
            def _():
                fire_gather(c + 1, 1 - slot)

            store_desc(c, slot).start()
            return carry

        lax.fori_loop(0, n_chunks, chunk_body, 0)
        store_desc(n_chunks - 1, lax.rem(n_chunks - 1, 2)).wait()

    return body(x_flat, table)


def kernel(x, raw_weight):
    B, S = x.shape
    xp = jnp.pad(x.astype(jnp.int32), ((0, 0), (0, SP - S))).reshape(-1)
    return _sc_embedding_gather(xp, raw_weight, S, B)


# speedup vs baseline: 1.4776x; 1.0246x over previous
import functools

import jax
import jax.numpy as jnp
from jax import lax
from jax.experimental import pallas as pl
from jax.experimental.pallas import tpu as pltpu
from jax.experimental.pallas import tpu_sc as plsc

EMBED_DIM = 32
NUM_WORKERS = 32
CHUNK_B = 16


def _sc_embedding_gather(x, table):
    B, S = x.shape
    b_per_w = B // NUM_WORKERS
    n_chunks = b_per_w // CHUNK_B

    mesh = plsc.VectorSubcoreMesh(core_axis_name="c", subcore_axis_name="s")

    @functools.partial(
        pl.kernel,
        out_type=jax.ShapeDtypeStruct((B, S, EMBED_DIM), jnp.float32),
        mesh=mesh,
        scratch_types=[
            pltpu.VMEM((b_per_w, S), jnp.int32),
            pltpu.VMEM((2, CHUNK_B, S, EMBED_DIM), jnp.float32),
            pltpu.SemaphoreType.DMA,
            pltpu.SemaphoreType.DMA,
        ],
        compiler_params=pltpu.CompilerParams(use_tc_tiling_on_sc=False),
    )
    def body(x_hbm, tab_hbm, out_hbm, idx_v, rows_v, gsem, ssem):
        wid = lax.axis_index("s") * 2 + lax.axis_index("c")
        base = wid * b_per_w
        pltpu.sync_copy(x_hbm.at[pl.ds(base, b_per_w)], idx_v)

        def fire_gather(c, slot):
            for r in range(CHUNK_B):
                pltpu.make_async_copy(
                    tab_hbm.at[idx_v.at[c * CHUNK_B + r]],
                    rows_v.at[slot, r],
                    gsem,
                ).start()

        def wait_gather(c, slot):
            for r in range(CHUNK_B):
                pltpu.make_async_copy(
                    tab_hbm.at[idx_v.at[c * CHUNK_B + r]],
                    rows_v.at[slot, r],
                    gsem,
                ).wait()

        def store_desc(c, slot):
            return pltpu.make_async_copy(
                rows_v.at[slot],
                out_hbm.at[pl.ds(base + c * CHUNK_B, CHUNK_B)],
                ssem,
            )

        fire_gather(0, 0)

        def chunk_body(c, carry):
            slot = lax.rem(c, 2)
            wait_gather(c, slot)

            @pl.when(c >= 1)
            def _():
                store_desc(c - 1, 1 - slot).wait()

            @pl.when(c + 1 < n_chunks)
            def _():
                fire_gather(c + 1, 1 - slot)

            store_desc(c, slot).start()
            return carry

        lax.fori_loop(0, n_chunks, chunk_body, 0)
        store_desc(n_chunks - 1, lax.rem(n_chunks - 1, 2)).wait()

    return body(x, table)


def kernel(x, raw_weight):
    return _sc_embedding_gather(x.astype(jnp.int32), raw_weight)
